# baseline (speedup 1.0000x reference)
import functools

import jax
import jax.numpy as jnp
from jax import lax
from jax.experimental import pallas as pl
from jax.experimental.pallas import tpu as pltpu
from jax.experimental.pallas import tpu_sc as plsc

_VOCAB = 1000000
_D = 16
_B = 16384
_HIST = 200

_NC = 2
_NS = 16
_NW = _NC * _NS

_LT = _HIST // 8
_BT = _B // 128
_BTW = _BT // _NW
_ROWS = 8 * 128

_VG = 8
_VTW = 248
_VT = _NW * _VTW
_VP = _VT * 128


def _cbody(cb4_hbm, rows_hbm, inb, r_v, isem, osem):
    wid = lax.axis_index("s") * _NC + lax.axis_index("c")
    lanes = lax.iota(jnp.int32, 16)
    dt_vec = lanes // 8
    din_vec = lanes % 8

    def fetch(vt, p):
        pltpu.async_copy(cb4_hbm.at[0, pl.ds(vt, _VG)], inb[p].at[0], isem[p])
        pltpu.async_copy(cb4_hbm.at[1, pl.ds(vt, _VG)], inb[p].at[1], isem[p])

    def drain_fetch(p):
        pltpu.make_async_copy(cb4_hbm.at[0, pl.ds(0, _VG)], inb[p].at[0],
                              isem[p]).wait()
        pltpu.make_async_copy(cb4_hbm.at[1, pl.ds(0, _VG)], inb[p].at[1],
                              isem[p]).wait()

    def transpose(p):
        @pl.loop(0, _VG * 128, step=16)
        def _vin(j0):
            t = j0 // 128
            vin0 = j0 % 128
            tsplat = jnp.full((16,), t, dtype=jnp.int32)
            vals = [
                plsc.load_gather(
                    inb[p],
                    [dt_vec, tsplat, din_vec,
                     jnp.full((16,), vin0 + k, dtype=jnp.int32)],
                )
                for k in range(16)
            ]
            for k in range(16):
                r_v[p][j0 + k] = vals[k]

    def store(vt, p):
        pltpu.async_copy(
            r_v[p], rows_hbm.at[pl.ds(vt * 128, _VG * 128)], osem[p]
        )

    def wait_store(p):
        pltpu.make_async_copy(
            r_v[p], rows_hbm.at[pl.ds(0, _VG * 128)], osem[p]
        ).wait()

    vt0 = wid * _VTW
    fetch(vt0, 0)

    @pl.loop(0, _VTW, step=2 * _VG)
    def _pair(k):
        vt = vt0 + k
        fetch(vt + _VG, 1)
        drain_fetch(0)

        @pl.when(k > 0)
        def _():
            wait_store(0)

        transpose(0)
        store(vt, 0)

        @pl.when(k + 2 * _VG < _VTW)
        def _():
            fetch(vt + 2 * _VG, 0)

        drain_fetch(1)

        @pl.when(k > 0)
        def _():
            wait_store(1)

        transpose(1)
        store(vt + _VG, 1)

    wait_store(0)
    wait_store(1)


def _body(x_hbm, cb_hbm, out_hbm, idx_v, rows_v, t_v, iqsem, gsem, osem):
    wid = lax.axis_index("s") * _NC + lax.axis_index("c")
    lanes = lax.iota(jnp.int32, 16)

    def idx_fetch(c, q):
        lt = c % _LT
        bt = wid * _BTW + c // _LT
        pltpu.async_copy(x_hbm.at[lt, bt], idx_v[q], iqsem[q])

    def fires(c, p):
        pltpu.make_async_copy(x_hbm.at[0, 0], idx_v[p], iqsem[p]).wait()
        for j in range(8):
            pltpu.async_copy(
                cb_hbm.at[idx_v[p].at[j]], rows_v[p].at[j], gsem[p]
            )

    def drain_gathers(p):
        for _ in range(8):
            pltpu.make_async_copy(
                cb_hbm.at[idx_v[p].at[0]], rows_v[p].at[0], gsem[p]
            ).wait()

    def transpose(pr, pt):
        @pl.loop(0, _D)
        def _dim(d):
            dsplat = jnp.full((16,), d, dtype=jnp.int32)
            for lin in range(8):
                lsplat = jnp.full((16,), lin, dtype=jnp.int32)
                vals = [
                    plsc.load_gather(
                        rows_v[pr],
                        [lsplat, lanes + bg * 16, dsplat],
                    )
                    for bg in range(8)
                ]
                for bg in range(8):
                    t_v[pt][d, lin, pl.ds(bg * 16, 16)] = vals[bg]

    def store(c, p):
        lt = c % _LT
        bt = wid * _BTW + c // _LT
        pltpu.async_copy(t_v[p], out_hbm.at[:, lt, bt], osem[p])

    def wait_store_all(p):
        pltpu.make_async_copy(t_v[p], out_hbm.at[:, 0, 0], osem[p]).wait()

    n_chunks = _LT * _BTW

    idx_fetch(0, 0)
    idx_fetch(1, 1)
    idx_fetch(2, 2)
    fires(0, 0)
    fires(1, 1)

    @pl.loop(0, n_chunks, step=4)
    def _quad(c):
        for j in range(4):
            @pl.when(c + j + 3 < n_chunks)
            def _():
                idx_fetch(c + j + 3, (j + 3) % 4)

            @pl.when(c + j + 2 < n_chunks)
            def _():
                fires(c + j + 2, (j + 2) % 4)

            drain_gathers(j)

            @pl.when(c + j >= 2)
            def _():
                wait_store_all(j % 2)

            transpose(j, j % 2)
            store(c + j, j % 2)

    wait_store_all(0)
    wait_store_all(1)


@jax.jit
def _run(x_t, codebook):
    mesh = plsc.VectorSubcoreMesh(core_axis_name="c", subcore_axis_name="s")
    params = pltpu.CompilerParams(
        needs_layout_passes=False, use_tc_tiling_on_sc=False
    )

    cbp = jnp.pad(codebook, ((0, _VP - _VOCAB), (0, 0)))
    cb4 = cbp.T.reshape(2, 8, _VT, 128).transpose(0, 2, 1, 3)

    conv = pl.kernel(
        _cbody,
        out_type=jax.ShapeDtypeStruct((_VP, _D), jnp.float32),
        mesh=mesh,
        scratch_types=[
            [pltpu.VMEM((2, _VG, 8, 128), jnp.float32)] * 2,
            [pltpu.VMEM((_VG * 128, _D), jnp.float32)] * 2,
            [pltpu.SemaphoreType.DMA] * 2,
            [pltpu.SemaphoreType.DMA] * 2,
        ],
        compiler_params=params,
    )
    cb_rows = conv(cb4)

    f = pl.kernel(
        _body,
        out_type=jax.ShapeDtypeStruct((_D, _LT, _BT, 8, 128), jnp.float32),
        mesh=mesh,
        scratch_types=[
            [pltpu.VMEM((8, 128), jnp.int32)] * 4,
            [pltpu.VMEM((8, 128, _D), jnp.float32)] * 4,
            [pltpu.VMEM((_D, 8, 128), jnp.float32)] * 2,
            [pltpu.SemaphoreType.DMA] * 4,
            [pltpu.SemaphoreType.DMA] * 4,
            [pltpu.SemaphoreType.DMA] * 2,
        ],
        compiler_params=params,
    )
    return f(x_t, cb_rows)


def kernel(x, codebook):
    x_t = x.T.reshape(_LT, 8, _BT, 128).transpose(0, 2, 1, 3)
    out5 = _run(x_t, codebook)
    out = (
        out5.transpose(0, 1, 3, 2, 4)
        .reshape(_D, _HIST, _B)
        .transpose(2, 0, 1)
    )
    return out


# speedup vs baseline: 1.0095x; 1.0095x over previous
import functools

import jax
import jax.numpy as jnp
from jax import lax
from jax.experimental import pallas as pl
from jax.experimental.pallas import tpu as pltpu
from jax.experimental.pallas import tpu_sc as plsc

_VOCAB = 1000000
_D = 16
_B = 16384
_HIST = 200

_NC = 2
_NS = 16
_NW = _NC * _NS

_LT = _HIST // 8
_BT = _B // 128
_BTW = _BT // _NW
_ROWS = 8 * 128

_VG = 4
_VTW = 248
_VT = _NW * _VTW
_VP = _VT * 128


def _cbody(cb4_hbm, rows_hbm, inb, r_v, isem, osem):
    wid = lax.axis_index("s") * _NC + lax.axis_index("c")
    lanes = lax.iota(jnp.int32, 16)
    dt_vec = lanes // 8
    din_vec = lanes % 8

    def fetch(vt, p):
        pltpu.async_copy(cb4_hbm.at[0, pl.ds(vt, _VG)], inb[p].at[0], isem[p])
        pltpu.async_copy(cb4_hbm.at[1, pl.ds(vt, _VG)], inb[p].at[1], isem[p])

    def drain_fetch(p):
        pltpu.make_async_copy(cb4_hbm.at[0, pl.ds(0, _VG)], inb[p].at[0],
                              isem[p]).wait()
        pltpu.make_async_copy(cb4_hbm.at[1, pl.ds(0, _VG)], inb[p].at[1],
                              isem[p]).wait()

    def transpose(p):
        @pl.loop(0, _VG * 128, step=16)
        def _vin(j0):
            t = j0 // 128
            vin0 = j0 % 128
            tsplat = jnp.full((16,), t, dtype=jnp.int32)
            vals = [
                plsc.load_gather(
                    inb[p],
                    [dt_vec, tsplat, din_vec,
                     jnp.full((16,), vin0 + k, dtype=jnp.int32)],
                )
                for k in range(16)
            ]
            for k in range(16):
                r_v[p][j0 + k] = vals[k]

    def store(vt, p):
        pltpu.async_copy(
            r_v[p], rows_hbm.at[pl.ds(vt * 128, _VG * 128)], osem[p]
        )

    def wait_store(p):
        pltpu.make_async_copy(
            r_v[p], rows_hbm.at[pl.ds(0, _VG * 128)], osem[p]
        ).wait()

    vt0 = wid * _VTW
    fetch(vt0, 0)

    @pl.loop(0, _VTW, step=2 * _VG)
    def _pair(k):
        vt = vt0 + k
        fetch(vt + _VG, 1)
        drain_fetch(0)

        @pl.when(k > 0)
        def _():
            wait_store(0)

        transpose(0)
        store(vt, 0)

        @pl.when(k + 2 * _VG < _VTW)
        def _():
            fetch(vt + 2 * _VG, 0)

        drain_fetch(1)

        @pl.when(k > 0)
        def _():
            wait_store(1)

        transpose(1)
        store(vt + _VG, 1)

    wait_store(0)
    wait_store(1)


def _body(x_hbm, cb_hbm, out_hbm, idx_v, rows_v, t_v, iqsem, gsem, osem):
    wid = lax.axis_index("s") * _NC + lax.axis_index("c")
    lanes = lax.iota(jnp.int32, 16)

    def idx_fetch(c, q):
        lt = c % _LT
        bt = wid * _BTW + c // _LT
        pltpu.async_copy(x_hbm.at[lt, bt], idx_v[q], iqsem[q])

    def fires(c, p):
        pltpu.make_async_copy(x_hbm.at[0, 0], idx_v[p], iqsem[p]).wait()
        for j in range(8):
            pltpu.async_copy(
                cb_hbm.at[idx_v[p].at[j]], rows_v[p].at[j], gsem[p]
            )

    def drain_gathers(p):
        for _ in range(8):
            pltpu.make_async_copy(
                cb_hbm.at[idx_v[p].at[0]], rows_v[p].at[0], gsem[p]
            ).wait()

    def transpose(pr, pt):
        @pl.loop(0, _D)
        def _dim(d):
            dsplat = jnp.full((16,), d, dtype=jnp.int32)
            for lin in range(8):
                lsplat = jnp.full((16,), lin, dtype=jnp.int32)
                vals = [
                    plsc.load_gather(
                        rows_v[pr],
                        [lsplat, lanes + bg * 16, dsplat],
                    )
                    for bg in range(8)
                ]
                for bg in range(8):
                    t_v[pt][d, lin, pl.ds(bg * 16, 16)] = vals[bg]

    def store(c, p):
        lt = c % _LT
        bt = wid * _BTW + c // _LT
        pltpu.async_copy(t_v[p], out_hbm.at[:, lt, bt], osem[p])

    def wait_store_all(p):
        pltpu.make_async_copy(t_v[p], out_hbm.at[:, 0, 0], osem[p]).wait()

    n_chunks = _LT * _BTW

    idx_fetch(0, 0)
    idx_fetch(1, 1)
    idx_fetch(2, 2)
    fires(0, 0)
    fires(1, 1)

    @pl.loop(0, n_chunks, step=4)
    def _quad(c):
        for j in range(4):
            @pl.when(c + j + 3 < n_chunks)
            def _():
                idx_fetch(c + j + 3, (j + 3) % 4)

            @pl.when(c + j + 2 < n_chunks)
            def _():
                fires(c + j + 2, (j + 2) % 4)

            drain_gathers(j)

            @pl.when(c + j >= 2)
            def _():
                wait_store_all(j % 2)

            transpose(j, j % 2)
            store(c + j, j % 2)

    wait_store_all(0)
    wait_store_all(1)


@jax.jit
def _run(x_t, codebook):
    mesh = plsc.VectorSubcoreMesh(core_axis_name="c", subcore_axis_name="s")
    params = pltpu.CompilerParams(
        needs_layout_passes=False, use_tc_tiling_on_sc=False
    )

    cbp = jnp.pad(codebook, ((0, _VP - _VOCAB), (0, 0)))
    cb4 = cbp.T.reshape(2, 8, _VT, 128).transpose(0, 2, 1, 3)

    conv = pl.kernel(
        _cbody,
        out_type=jax.ShapeDtypeStruct((_VP, _D), jnp.float32),
        mesh=mesh,
        scratch_types=[
            [pltpu.VMEM((2, _VG, 8, 128), jnp.float32)] * 2,
            [pltpu.VMEM((_VG * 128, _D), jnp.float32)] * 2,
            [pltpu.SemaphoreType.DMA] * 2,
            [pltpu.SemaphoreType.DMA] * 2,
        ],
        compiler_params=params,
    )
    cb_rows = conv(cb4)

    f = pl.kernel(
        _body,
        out_type=jax.ShapeDtypeStruct((_D, _LT, _BT, 8, 128), jnp.float32),
        mesh=mesh,
        scratch_types=[
            [pltpu.VMEM((8, 128), jnp.int32)] * 4,
            [pltpu.VMEM((8, 128, _D), jnp.float32)] * 4,
            [pltpu.VMEM((_D, 8, 128), jnp.float32)] * 2,
            [pltpu.SemaphoreType.DMA] * 4,
            [pltpu.SemaphoreType.DMA] * 4,
            [pltpu.SemaphoreType.DMA] * 2,
        ],
        compiler_params=params,
    )
    return f(x_t, cb_rows)


def kernel(x, codebook):
    x_t = x.T.reshape(_LT, 8, _BT, 128).transpose(0, 2, 1, 3)
    out5 = _run(x_t, codebook)
    out = (
        out5.transpose(0, 1, 3, 2, 4)
        .reshape(_D, _HIST, _B)
        .transpose(2, 0, 1)
    )
    return out
